# baseline (speedup 1.0000x reference)
import jax
import jax.numpy as jnp
from jax import lax
from jax.experimental import pallas as pl
from jax.experimental.pallas import tpu as pltpu
from jax.experimental.pallas import tpu_sc as plsc

VOCAB = 100000
DIM = 128
BATCH = 4096
SEQ = 200

_info = plsc.get_sparse_core_info()
_NC, _NS = _info.num_cores, _info.num_subcores
NW = _NC * _NS

B = BATCH * SEQ
B_PER_W = B // NW
CHUNK = 128
NCHUNK = B_PER_W // CHUNK
NBUF = 2


def _gather_body(x_hbm, table_hbm, out_hbm, idx_v,
                 rows0, rows1, spm,
                 gsem0, gsem1, csem0, csem1, dsem0, dsem1):
    rows = (rows0, rows1)
    gsems = (gsem0, gsem1)
    csems = (csem0, csem1)
    dsems = (dsem0, dsem1)
    sid = lax.axis_index("s")
    wid = sid * _NC + lax.axis_index("c")
    pltpu.sync_copy(x_hbm.at[wid], idx_v)
    base = wid * B_PER_W

    def fire_g(j, b):
        pltpu.async_copy(table_hbm.at[idx_v.at[j]], rows[b], gsems[b])

    def wait_g(j, b):
        pltpu.make_async_copy(
            table_hbm.at[idx_v.at[j]], rows[b], gsems[b]).wait()

    for b in range(NBUF):
        fire_g(b, b)

    def outer(jo, carry):
        for b in range(NBUF):
            j = jo * NBUF + b
            my_spm = spm.at[sid, b]
            wait_g(j, b)

            def wait_d():
                pltpu.make_async_copy(
                    my_spm, out_hbm.at[pl.ds(base, CHUNK)], dsems[b]).wait()
            pl.when(jo > 0)(wait_d)

            pltpu.async_copy(rows[b], my_spm, csems[b])
            pltpu.make_async_copy(rows[b], my_spm, csems[b]).wait()
            pltpu.async_copy(
                my_spm, out_hbm.at[pl.ds(base + j * CHUNK, CHUNK)], dsems[b])

            def next_g():
                fire_g(j + NBUF, b)
            pl.when(jo < (NCHUNK // NBUF) - 1)(next_g)
        return carry

    lax.fori_loop(0, NCHUNK // NBUF, outer, 0)

    for b in range(NBUF):
        pltpu.make_async_copy(
            spm.at[sid, b], out_hbm.at[pl.ds(base, CHUNK)], dsems[b]).wait()


def kernel(x, table):
    mesh = plsc.VectorSubcoreMesh(core_axis_name="c", subcore_axis_name="s")
    x_blocks = x.reshape(NW, NCHUNK, CHUNK).astype(jnp.int32)
    flat = pl.kernel(
        _gather_body,
        out_type=jax.ShapeDtypeStruct((B, DIM), jnp.float32),
        mesh=mesh,
        scratch_types=(
            [pltpu.VMEM((NCHUNK, CHUNK), jnp.int32)]
            + [pltpu.VMEM((CHUNK, DIM), jnp.float32)] * NBUF
            + [pltpu.VMEM_SHARED((_NS, NBUF, CHUNK, DIM), jnp.float32)]
            + [pltpu.SemaphoreType.DMA] * (3 * NBUF)
        ),
    )(x_blocks, table)
    return flat.reshape(BATCH, SEQ, DIM)
